# baseline (speedup 1.0000x reference)
import functools

import jax
import jax.numpy as jnp
from jax import lax
from jax.experimental import pallas as pl
from jax.experimental.pallas import tpu as pltpu
from jax.experimental.pallas import tpu_sc as plsc

_NC = 2
_NS = 16
_NW = _NC * _NS
_LANES = 16
_SUB = 128


def _sc_gather_scale(x_r, table, scale16, *, n_rows, d, g_chunks, k_subs):
    chunk = k_subs * _SUB
    per_w = g_chunks * chunk
    vregs_per_row = d // _LANES

    mesh = plsc.VectorSubcoreMesh(core_axis_name="c", subcore_axis_name="s")

    @functools.partial(
        pl.kernel,
        out_type=jax.ShapeDtypeStruct((n_rows, d), jnp.float32),
        mesh=mesh,
        scratch_types=[
            pltpu.VMEM((g_chunks, k_subs, _SUB), jnp.int32),
            pltpu.VMEM((2, chunk, d), jnp.float32),
            pltpu.VMEM((_LANES,), jnp.float32),
            pltpu.SemaphoreType.DMA((2,)),
            pltpu.SemaphoreType.DMA((2,)),
        ],
        compiler_params=pltpu.CompilerParams(use_tc_tiling_on_sc=False),
    )
    def k(x_hbm, table_hbm, scale_hbm, out_hbm, idx_v, rows_v, scale_v, gsem, osem):
        wid = lax.axis_index("s") * _NC + lax.axis_index("c")
        base = wid * per_w

        pltpu.sync_copy(scale_hbm, scale_v)
        pltpu.sync_copy(x_hbm.at[wid], idx_v)
        s = scale_v[...]

        def issue_gathers(g, b):
            return [
                pltpu.async_copy(
                    table_hbm.at[idx_v.at[g, j]],
                    rows_v.at[b, pl.ds(j * _SUB, _SUB)],
                    gsem.at[b],
                )
                for j in range(k_subs)
            ]

        gather_h = [None, None]
        scatter_h = [None, None]
        gather_h[0] = issue_gathers(0, 0)
        for g in range(g_chunks):
            b = g % 2
            nb = 1 - b
            if g + 1 < g_chunks:
                if scatter_h[nb] is not None:
                    scatter_h[nb].wait()
                gather_h[nb] = issue_gathers(g + 1, nb)
            for h in gather_h[b]:
                h.wait()

            def mul_body(i, _, b=b):
                for c in range(vregs_per_row):
                    sl = pl.ds(c * _LANES, _LANES)
                    rows_v[b, i, sl] = rows_v[b, i, sl] * s
                return 0

            lax.fori_loop(0, chunk, mul_body, 0)
            scatter_h[b] = pltpu.async_copy(
                rows_v.at[b],
                out_hbm.at[pl.ds(base + g * chunk, chunk)],
                osem.at[b],
            )
        scatter_h[0].wait()
        if scatter_h[1] is not None:
            scatter_h[1].wait()

    return k(x_r, table, scale16)


_PREP_ROWS = 4000
_PREP_HALF = _PREP_ROWS // 2
_PREP_SUP = 8


def _remap(y):
    j8 = (y // _PREP_ROWS) * _PREP_ROWS
    r = y - j8
    return j8 + 2 * r - jnp.where(r >= _PREP_HALF, _PREP_ROWS - 1, 0)


def _tc_transpose(xt):
    l, b = xt.shape

    def body(src, dst):
        dst[...] = _remap(src[...].T)

    return pl.pallas_call(
        body,
        out_shape=jax.ShapeDtypeStruct((b, l), jnp.int32),
    )(xt)


def _tc_table_linearize(t_cm):
    d, v = t_cm.shape
    assert v % _PREP_ROWS == 0 and 2 * d == 128
    units = v // _PREP_ROWS
    grid = (units + _PREP_SUP - 1) // _PREP_SUP
    t3 = t_cm.reshape(d, units, _PREP_ROWS)
    ublk = _PREP_ROWS * d // 128

    def body(src, dst):
        parts = []
        for k in range(_PREP_SUP):
            tr = src[:, k, :].T
            parts.append(jnp.concatenate(
                [tr[:_PREP_HALF, :], tr[_PREP_HALF:, :]], axis=1))
        dst[...] = jnp.concatenate(parts, axis=0)

    return pl.pallas_call(
        body,
        grid=(grid,),
        in_specs=[pl.BlockSpec((d, _PREP_SUP, _PREP_ROWS),
                               lambda j: (0, j, 0))],
        out_specs=pl.BlockSpec((_PREP_SUP * ublk, 128), lambda j: (j, 0)),
        out_shape=jax.ShapeDtypeStruct((v * d // 128, 128), jnp.float32),
    )(t3)


def kernel(x, table, scale):
    b, l = x.shape
    v, d = table.shape
    n = b * l
    assert d % _LANES == 0
    assert n % (_NW * _SUB) == 0
    subs_per_w = n // (_NW * _SUB)
    k_subs = 5 if subs_per_w % 5 == 0 else 1
    g_chunks = subs_per_w // k_subs

    x_rm = _tc_transpose(x.astype(jnp.int32).T)
    x_r = x_rm.reshape(_NW, g_chunks, k_subs, _SUB)
    scale16 = jnp.broadcast_to(scale.astype(jnp.float32), (_LANES,))
    t_rm = _tc_table_linearize(table.astype(jnp.float32).T).reshape(v, d)
    out = _sc_gather_scale(
        x_r, t_rm, scale16,
        n_rows=n, d=d, g_chunks=g_chunks, k_subs=k_subs,
    )
    return out.reshape(b, l, d)


# speedup vs baseline: 1.6706x; 1.6706x over previous
import functools

import jax
import jax.numpy as jnp
from jax import lax
from jax.experimental import pallas as pl
from jax.experimental.pallas import tpu as pltpu
from jax.experimental.pallas import tpu_sc as plsc

_NC = 2
_NS = 16
_NW = _NC * _NS
_LANES = 16
_SUB = 128


def _sc_gather_scale(x_r, table, scale16, *, n_rows, d, g_chunks, k_subs):
    chunk = k_subs * _SUB
    per_w = g_chunks * chunk
    vregs_per_row = d // _LANES

    mesh = plsc.VectorSubcoreMesh(core_axis_name="c", subcore_axis_name="s")

    @functools.partial(
        pl.kernel,
        out_type=jax.ShapeDtypeStruct((n_rows, d), jnp.float32),
        mesh=mesh,
        scratch_types=[
            pltpu.VMEM((g_chunks, k_subs, _SUB), jnp.int32),
            pltpu.VMEM((2, chunk, d), jnp.float32),
            pltpu.VMEM((_LANES,), jnp.float32),
            pltpu.SemaphoreType.DMA((2,)),
            pltpu.SemaphoreType.DMA((2,)),
        ],
        compiler_params=pltpu.CompilerParams(use_tc_tiling_on_sc=False),
    )
    def k(x_hbm, table_hbm, scale_hbm, out_hbm, idx_v, rows_v, scale_v, gsem, osem):
        wid = lax.axis_index("s") * _NC + lax.axis_index("c")
        base = wid * per_w

        pltpu.sync_copy(scale_hbm, scale_v)
        pltpu.sync_copy(x_hbm.at[wid], idx_v)
        s = scale_v[...]

        def issue_gathers(g, b):
            return [
                pltpu.async_copy(
                    table_hbm.at[idx_v.at[g, j]],
                    rows_v.at[b, pl.ds(j * _SUB, _SUB)],
                    gsem.at[b],
                )
                for j in range(k_subs)
            ]

        gather_h = [None, None]
        scatter_h = [None, None]
        gather_h[0] = issue_gathers(0, 0)
        for g in range(g_chunks):
            b = g % 2
            nb = 1 - b
            if g + 1 < g_chunks:
                if scatter_h[nb] is not None:
                    scatter_h[nb].wait()
                gather_h[nb] = issue_gathers(g + 1, nb)
            for h in gather_h[b]:
                h.wait()

            def mul_body(i, _, b=b):
                for c in range(vregs_per_row):
                    sl = pl.ds(c * _LANES, _LANES)
                    rows_v[b, i, sl] = rows_v[b, i, sl] * s
                return 0

            lax.fori_loop(0, chunk, mul_body, 0)
            scatter_h[b] = pltpu.async_copy(
                rows_v.at[b],
                out_hbm.at[pl.ds(base + g * chunk, chunk)],
                osem.at[b],
            )
        scatter_h[0].wait()
        if scatter_h[1] is not None:
            scatter_h[1].wait()

    return k(x_r, table, scale16)


_PREP_ROWS = 4096
_PREP_HALF = _PREP_ROWS // 2


def _remap(y):
    j8 = (y // _PREP_ROWS) * _PREP_ROWS
    r = y - j8
    return j8 + 2 * r - jnp.where(r >= _PREP_HALF, _PREP_ROWS - 1, 0)


def _tc_transpose(xt):
    l, b = xt.shape

    def body(src, dst):
        dst[...] = _remap(src[...].T)

    return pl.pallas_call(
        body,
        out_shape=jax.ShapeDtypeStruct((b, l), jnp.int32),
    )(xt)


def _tc_table_linearize(t_cm):
    d, v = t_cm.shape
    assert 2 * d == 128
    grid = (v + _PREP_ROWS - 1) // _PREP_ROWS
    v_pad = grid * _PREP_ROWS

    def body(src, dst):
        tr = src[...].T
        dst[...] = jnp.concatenate(
            [tr[:_PREP_HALF, :], tr[_PREP_HALF:, :]], axis=1)

    out = pl.pallas_call(
        body,
        grid=(grid,),
        in_specs=[pl.BlockSpec((d, _PREP_ROWS), lambda j: (0, j))],
        out_specs=pl.BlockSpec((_PREP_HALF, 128), lambda j: (j, 0)),
        out_shape=jax.ShapeDtypeStruct((v_pad * d // 128, 128),
                                       jnp.float32),
    )(t_cm)
    return out, v_pad


def kernel(x, table, scale):
    b, l = x.shape
    v, d = table.shape
    n = b * l
    assert d % _LANES == 0
    assert n % (_NW * _SUB) == 0
    subs_per_w = n // (_NW * _SUB)
    k_subs = 5 if subs_per_w % 5 == 0 else 1
    g_chunks = subs_per_w // k_subs

    x_rm = _tc_transpose(x.astype(jnp.int32).T)
    x_r = x_rm.reshape(_NW, g_chunks, k_subs, _SUB)
    scale16 = jnp.broadcast_to(scale.astype(jnp.float32), (_LANES,))
    t_packed, v_pad = _tc_table_linearize(table.astype(jnp.float32).T)
    t_rm = t_packed.reshape(v_pad, d)
    out = _sc_gather_scale(
        x_r, t_rm, scale16,
        n_rows=n, d=d, g_chunks=g_chunks, k_subs=k_subs,
    )
    return out.reshape(b, l, d)
